# Initial kernel scaffold; baseline (speedup 1.0000x reference)
import jax
import jax.numpy as jnp
from jax.experimental import pallas as pl


def kernel(hm, wh, reg):
    raise NotImplementedError("write your pallas kernel here")


# speedup vs baseline: 10.8328x; 10.8328x over previous
import functools

import jax
import jax.numpy as jnp
from jax import lax
from jax.experimental import pallas as pl
from jax.experimental.pallas import tpu as pltpu

_DOWN_RATIO = 4.0
_K = 100
_BIG = 2**30


def _decode_body(hm_ref, wh_ref, reg_ref, boxes_ref, scores_ref, cls_ref,
                 s_ref, m_ref, q_ref, *, C, H, W, K):
    h = hm_ref[0]
    heat = jnp.clip(jax.nn.sigmoid(h), 1e-4, 1.0 - 1e-4)

    neg = jnp.float32(-1.0)
    pad_w = jnp.full((C, H, 1), neg, jnp.float32)
    left = jnp.concatenate([pad_w, heat[:, :, : W - 1]], axis=2)
    right = jnp.concatenate([heat[:, :, 1:], pad_w], axis=2)
    hw = jnp.maximum(jnp.maximum(left, right), heat)
    pad_h = jnp.full((C, 1, W), neg, jnp.float32)
    up = jnp.concatenate([pad_h, hw[:, : H - 1, :]], axis=1)
    down = jnp.concatenate([hw[:, 1:, :], pad_h], axis=1)
    hmax = jnp.maximum(jnp.maximum(up, down), hw)

    sup = jnp.where(heat == hmax, heat, 0.0)
    s_ref[...] = sup
    m_ref[...] = jnp.max(sup, axis=2)
    q_ref[...] = jnp.zeros_like(q_ref)

    flat_ch = (lax.broadcasted_iota(jnp.int32, (C, H), 0) * H
               + lax.broadcasted_iota(jnp.int32, (C, H), 1))
    lane = lax.broadcasted_iota(jnp.int32, (1, 128), 1)
    lane3 = lax.broadcasted_iota(jnp.int32, (1, 1, W), 2)

    def body(k, carry):
        mv = m_ref[...]
        m = jnp.max(mv)
        idx = jnp.min(jnp.where(mv == m, flat_ch, _BIG))
        c = idx // H
        r = idx % H

        row = s_ref[pl.ds(c, 1), pl.ds(r, 1), :]
        col = jnp.min(jnp.where(row == m, lane3, _BIG))
        new_row = jnp.where(lane3 == col, neg, row)
        s_ref[pl.ds(c, 1), pl.ds(r, 1), :] = new_row
        new_max = jnp.max(new_row)

        mrow = m_ref[pl.ds(c, 1), :]
        m_ref[pl.ds(c, 1), :] = jnp.where(
            lax.broadcasted_iota(jnp.int32, (1, H), 1) == r, new_max, mrow)

        def pick(vrow):
            return jnp.sum(jnp.where(
                lax.broadcasted_iota(jnp.int32, (1, W), 1) == col, vrow, 0.0))

        reg0 = pick(reg_ref[0, 0, pl.ds(r, 1), :])
        reg1 = pick(reg_ref[0, 1, pl.ds(r, 1), :])
        wh0 = pick(wh_ref[0, 0, pl.ds(r, 1), :])
        wh1 = pick(wh_ref[0, 1, pl.ds(r, 1), :])

        def putq(qi, val):
            cur = q_ref[pl.ds(qi, 1), :]
            q_ref[pl.ds(qi, 1), :] = jnp.where(lane == k, val, cur)

        putq(0, m)
        putq(1, c.astype(jnp.float32))
        putq(2, r.astype(jnp.float32))
        putq(3, col.astype(jnp.float32))
        putq(4, reg0)
        putq(5, reg1)
        putq(6, wh0)
        putq(7, wh1)
        return carry

    lax.fori_loop(0, K, body, 0, unroll=False)

    q = q_ref[...]
    score = q[0:1, :K]
    clsv = q[1:2, :K]
    ys = q[2:3, :K] + q[5:6, :K]
    xs = q[3:4, :K] + q[4:5, :K]
    wv = q[6:7, :K]
    hv = q[7:8, :K]
    x1 = (xs - wv * 0.5) * _DOWN_RATIO
    y1 = (ys - hv * 0.5) * _DOWN_RATIO
    x2 = (xs + wv * 0.5) * _DOWN_RATIO
    y2 = (ys + hv * 0.5) * _DOWN_RATIO
    boxes_ref[...] = jnp.concatenate([x1, y1, x2, y2], axis=0)[None]
    scores_ref[...] = score[None]
    cls_ref[...] = clsv[None]


def kernel(hm, wh, reg):
    B, C, H, W = hm.shape
    K = _K
    body = functools.partial(_decode_body, C=C, H=H, W=W, K=K)
    boxes_t, scores, classes = pl.pallas_call(
        body,
        grid=(B,),
        in_specs=[
            pl.BlockSpec((1, C, H, W), lambda b: (b, 0, 0, 0)),
            pl.BlockSpec((1, 2, H, W), lambda b: (b, 0, 0, 0)),
            pl.BlockSpec((1, 2, H, W), lambda b: (b, 0, 0, 0)),
        ],
        out_specs=[
            pl.BlockSpec((1, 4, K), lambda b: (b, 0, 0)),
            pl.BlockSpec((1, 1, K), lambda b: (b, 0, 0)),
            pl.BlockSpec((1, 1, K), lambda b: (b, 0, 0)),
        ],
        out_shape=[
            jax.ShapeDtypeStruct((B, 4, K), jnp.float32),
            jax.ShapeDtypeStruct((B, 1, K), jnp.float32),
            jax.ShapeDtypeStruct((B, 1, K), jnp.float32),
        ],
        scratch_shapes=[
            pltpu.VMEM((C, H, W), jnp.float32),
            pltpu.VMEM((C, H), jnp.float32),
            pltpu.VMEM((8, 128), jnp.float32),
        ],
    )(hm, wh, reg)
    boxes = jnp.transpose(boxes_t, (0, 2, 1))
    return boxes, scores[:, 0, :], classes[:, 0, :]
